# baseline (speedup 1.0000x reference)
import functools

import jax
import jax.numpy as jnp
from jax import lax
from jax.experimental import pallas as pl
from jax.experimental.pallas import tpu as pltpu
from jax.experimental.pallas import tpu_sc as plsc

_D = 96
_DP = 128
_NC = 2
_NS = 16
_NW = _NC * _NS


def _sc_gather2(table, idx_a, idx_b, chunk):
    e = idx_a.shape[0]
    d = table.shape[1]
    bpw = e // _NW
    n_iter = bpw // chunk
    mesh = plsc.VectorSubcoreMesh(core_axis_name="c", subcore_axis_name="s")

    @functools.partial(
        pl.kernel,
        mesh=mesh,
        out_type=[
            jax.ShapeDtypeStruct((e, d), jnp.float32),
            jax.ShapeDtypeStruct((e, d), jnp.float32),
        ],
        scratch_types=[
            pltpu.VMEM((chunk,), jnp.int32),
            pltpu.VMEM((chunk,), jnp.int32),
            pltpu.VMEM((chunk, d), jnp.float32),
            pltpu.VMEM((chunk, d), jnp.float32),
            pltpu.SemaphoreType.DMA,
            pltpu.SemaphoreType.DMA,
        ],
    )
    def k(table_hbm, ia_hbm, ib_hbm, oa_hbm, ob_hbm,
          ia_v, ib_v, ra_v, rb_v, sem_a, sem_b):
        wid = lax.axis_index("s") * _NC + lax.axis_index("c")
        base = wid * bpw

        def body(i, _):
            off = base + i * chunk
            pltpu.sync_copy(ia_hbm.at[pl.ds(off, chunk)], ia_v)
            pltpu.sync_copy(ib_hbm.at[pl.ds(off, chunk)], ib_v)
            ca = pltpu.async_copy(table_hbm.at[ia_v], ra_v, sem_a)
            cb = pltpu.async_copy(table_hbm.at[ib_v], rb_v, sem_b)
            ca.wait()
            pltpu.sync_copy(ra_v, oa_hbm.at[pl.ds(off, chunk)])
            cb.wait()
            pltpu.sync_copy(rb_v, ob_hbm.at[pl.ds(off, chunk)])
            return 0

        lax.fori_loop(0, n_iter, body, 0)

    return k(table, idx_a, idx_b)


def _sc_gather1(table, idx, chunk):
    e = idx.shape[0]
    d = table.shape[1]
    bpw = e // _NW
    n_iter = bpw // chunk
    mesh = plsc.VectorSubcoreMesh(core_axis_name="c", subcore_axis_name="s")

    @functools.partial(
        pl.kernel,
        mesh=mesh,
        out_type=jax.ShapeDtypeStruct((e, d), jnp.float32),
        scratch_types=[
            pltpu.VMEM((chunk,), jnp.int32),
            pltpu.VMEM((chunk, d), jnp.float32),
            pltpu.SemaphoreType.DMA,
        ],
    )
    def k(table_hbm, i_hbm, o_hbm, i_v, r_v, sem):
        wid = lax.axis_index("s") * _NC + lax.axis_index("c")
        base = wid * bpw

        def body(i, _):
            off = base + i * chunk
            pltpu.sync_copy(i_hbm.at[pl.ds(off, chunk)], i_v)
            pltpu.async_copy(table_hbm.at[i_v], r_v, sem).wait()
            pltpu.sync_copy(r_v, o_hbm.at[pl.ds(off, chunk)])
            return 0

        lax.fori_loop(0, n_iter, body, 0)

    return k(table, idx)


_CN = 4352
_NCHUNK = 12
_G = 80
_BUF = 4096
_STOP = 3952
_NB = _BUF // _G


def _sc_segsum(sig_pad, bh, src, dst, e):
    n_out = _NCHUNK * _CN
    epw = e // _NS
    PF = 2000
    n_pf = epw // PF
    n_vec = PF // 16
    mesh = plsc.VectorSubcoreMesh(core_axis_name="c", subcore_axis_name="s")

    @functools.partial(
        pl.kernel,
        mesh=mesh,
        out_type=[
            jax.ShapeDtypeStruct((n_out, _DP), jnp.float32),
            jax.ShapeDtypeStruct((n_out, _DP), jnp.float32),
        ],
        scratch_types=[
            pltpu.VMEM((PF,), jnp.int32),
            pltpu.VMEM((PF,), jnp.int32),
            pltpu.VMEM((_BUF,), jnp.int32),
            pltpu.VMEM((_BUF,), jnp.int32),
            pltpu.VMEM((_BUF,), jnp.int32),
            pltpu.VMEM((_NB, _G), jnp.int32),
            pltpu.VMEM((_G, _DP), jnp.float32),
            pltpu.VMEM((_G, _DP), jnp.float32),
            pltpu.VMEM((_G, _DP), jnp.float32),
            pltpu.VMEM((_G, _DP), jnp.float32),
            pltpu.VMEM((16, _DP), jnp.float32),
            pltpu.VMEM_SHARED((_CN + 16, _DP), jnp.float32),
            pltpu.VMEM_SHARED((_CN + 16, _DP), jnp.float32),
            pltpu.SemaphoreType.DMA,
            pltpu.SemaphoreType.DMA,
            pltpu.SemaphoreType.DMA,
            pltpu.SemaphoreType.DMA,
        ],
    )
    def k(sig_hbm, bh_hbm, src_hbm, dst_hbm, num_hbm, den_hbm,
          dst_pf, src_pf, eid_b, src_b, dl_b, dl2d,
          sig0, sig1, bh0, bh1, zbuf,
          acc_num, acc_den, sem_s0, sem_s1, sem_b0, sem_b1):
        c = lax.axis_index("c")
        t = lax.axis_index("s")
        ebase = t * epw
        iota = lax.iota(jnp.int32, 16)

        def _z(r, _):
            for v in range(_DP // 16):
                zbuf[r, pl.ds(v * 16, 16)] = jnp.zeros((16,), jnp.float32)
            return 0
        lax.fori_loop(0, 16, _z, 0)

        flush_rows = _CN // _NS

        nbpf = PF // _G

        def one_pass(p, _):
            q = 2 * p + c
            lo = q * _CN
            hi = lo + _CN

            zb = t * flush_rows

            def zstep(zi, _):
                pltpu.sync_copy(zbuf, acc_num.at[pl.ds(zb + zi * 16, 16)])
                pltpu.sync_copy(zbuf, acc_den.at[pl.ds(zb + zi * 16, 16)])
                return 0

            lax.fori_loop(0, flush_rows // 16, zstep, 0)
            plsc.subcore_barrier()

            def pf_step(op, _):
                eb = ebase + op * PF
                pltpu.sync_copy(dst_hbm.at[pl.ds(eb, PF)], dst_pf)
                pltpu.sync_copy(src_hbm.at[pl.ds(eb, PF)], src_pf)

                def bstep(j, _):
                    for v in range(_G // 16):
                        sl = pl.ds(j * _G + v * 16, 16)
                        dv = dst_pf[sl]
                        m = (dv >= lo) & (dv < hi)
                        dl2d[0, pl.ds(v * 16, 16)] = jnp.where(
                            m, dv - lo, jnp.full((16,), _CN, jnp.int32))
                        src_b[pl.ds(v * 16, 16)] = src_pf[sl]
                    pltpu.sync_copy(sig_hbm.at[pl.ds(eb + j * _G, _G)], sig0)
                    pltpu.async_copy(
                        bh_hbm.at[src_b.at[pl.ds(0, _G)]], bh0, sem_b0).wait()

                    def mul_row(r, _):
                        for v in range(_DP // 16):
                            sl = pl.ds(v * 16, 16)
                            bh0[r, sl] = bh0[r, sl] * sig0[r, sl]
                        return 0
                    lax.fori_loop(0, _G, mul_row, 0)
                    pltpu.sync_copy(bh0, acc_num.at[dl2d.at[0]], add=True)
                    pltpu.sync_copy(sig0, acc_den.at[dl2d.at[0]], add=True)
                    return 0

                lax.fori_loop(0, nbpf, bstep, 0)
                return 0

            lax.fori_loop(0, n_pf, pf_step, 0)
            plsc.subcore_barrier()

            fb = t * flush_rows

            def fstep(fi, _):
                pltpu.sync_copy(acc_num.at[pl.ds(fb + fi * 16, 16)], zbuf)
                pltpu.sync_copy(zbuf, num_hbm.at[pl.ds(lo + fb + fi * 16, 16)])
                pltpu.sync_copy(acc_den.at[pl.ds(fb + fi * 16, 16)], zbuf)
                pltpu.sync_copy(zbuf, den_hbm.at[pl.ds(lo + fb + fi * 16, 16)])
                return 0

            lax.fori_loop(0, flush_rows // 16, fstep, 0)

            def _rz(r, _):
                for v in range(_DP // 16):
                    zbuf[r, pl.ds(v * 16, 16)] = jnp.zeros((16,), jnp.float32)
                return 0
            lax.fori_loop(0, 16, _rz, 0)
            plsc.subcore_barrier()
            return 0

        lax.fori_loop(0, _NCHUNK // _NC, one_pass, 0)

    return k(sig_pad, bh, src, dst)


def _ab_body(x_ref, wa_ref, ba_ref, wb_ref, bb_ref, ah_ref, bh_ref):
    x = x_ref[...]
    ah_ref[...] = jnp.dot(x, wa_ref[...], preferred_element_type=jnp.float32) + ba_ref[...]
    bh_ref[...] = jnp.dot(x, wb_ref[...], preferred_element_type=jnp.float32) + bb_ref[...]


def _sig_body(e_ref, s_ref):
    pad = jnp.zeros((e_ref.shape[0], _DP - _D), jnp.float32)
    s_ref[...] = jnp.concatenate(
        [jax.nn.sigmoid(e_ref[...]), pad], axis=1)


def _sig_pad(edge_feat, blk):
    m = edge_feat.shape[0]
    return pl.pallas_call(
        _sig_body,
        grid=(m // blk,),
        in_specs=[_row_spec(blk)],
        out_specs=pl.BlockSpec((blk, _DP), lambda i: (i, 0)),
        out_shape=jax.ShapeDtypeStruct((m, _DP), jnp.float32),
    )(edge_feat)


def _node_body(nf_ref, ah_ref, num_ref, den_ref, h_ref, stats_ref):
    i = pl.program_id(0)
    den = den_ref[...]
    pad = jnp.zeros((nf_ref.shape[0], _DP - _D), jnp.float32)
    h_agg = ah_ref[...] + num_ref[...] / (den + 1e-6)
    mask = den[:, 0:1] > 0.0
    h = jnp.where(mask, h_agg, jnp.concatenate([nf_ref[...], pad], axis=1))
    h_ref[...] = h

    @pl.when(i == 0)
    def _():
        stats_ref[...] = jnp.zeros_like(stats_ref)

    stats_ref[0:1, :] += jnp.sum(h, axis=0, keepdims=True)
    stats_ref[1:2, :] += jnp.sum(h * h, axis=0, keepdims=True)


def _gru_body(e_ref, sh_ref, dh_ref, w_ref, b_ref, e2_ref, stats_ref):
    i = pl.program_id(0)
    e = e_ref[...]
    sh = sh_ref[:, :_D]
    dh = dh_ref[:, :_D]

    def mm(x, k):
        return jnp.dot(x, w_ref[k], preferred_element_type=jnp.float32)

    def bias(k):
        return b_ref[k : k + 1, :]

    r1 = jax.nn.sigmoid(mm(sh, 0) + bias(0) + mm(e, 3) + bias(3))
    z1 = jax.nn.sigmoid(mm(sh, 1) + bias(1) + mm(e, 4) + bias(4))
    n1 = jnp.tanh(mm(sh, 2) + bias(2) + r1 * (mm(e, 5) + bias(5)))
    e1 = (1.0 - z1) * n1 + z1 * e

    r2 = jax.nn.sigmoid(mm(dh, 6) + bias(6) + mm(e1, 9) + bias(9))
    z2 = jax.nn.sigmoid(mm(dh, 7) + bias(7) + mm(e1, 10) + bias(10))
    n2 = jnp.tanh(mm(dh, 8) + bias(8) + r2 * (mm(e1, 11) + bias(11)))
    e2 = (1.0 - z2) * n2 + z2 * e1

    e2_ref[...] = e2

    @pl.when(i == 0)
    def _():
        stats_ref[...] = jnp.zeros_like(stats_ref)

    stats_ref[0:1, :] += jnp.sum(e2, axis=0, keepdims=True)
    stats_ref[1:2, :] += jnp.sum(e2 * e2, axis=0, keepdims=True)


def _apply_body(x_ref, res_ref, scale_ref, shift_ref, o_ref):
    x = x_ref[:, :_D]
    o_ref[...] = res_ref[...] + jnp.maximum(
        x * scale_ref[...] + shift_ref[...], 0.0
    )


def _full_spec():
    return pl.BlockSpec(lambda i: (0, 0))


def _row_spec(blk):
    return pl.BlockSpec((blk, _D), lambda i: (i, 0))


def _compute_ab(node_feat, W_A, b_A, W_B, b_B, blk):
    n = node_feat.shape[0]
    grid = n // blk
    padw = jnp.zeros((_D, _DP - _D), jnp.float32)
    padb = jnp.zeros((1, _DP - _D), jnp.float32)
    wa = jnp.concatenate([W_A.T, padw], axis=1)
    wb = jnp.concatenate([W_B.T, padw], axis=1)
    ba = jnp.concatenate([b_A.reshape(1, _D), padb], axis=1)
    bb = jnp.concatenate([b_B.reshape(1, _D), padb], axis=1)
    return pl.pallas_call(
        _ab_body,
        grid=(grid,),
        in_specs=[
            _row_spec(blk),
            pl.BlockSpec((_D, _DP), lambda i: (0, 0)),
            pl.BlockSpec((1, _DP), lambda i: (0, 0)),
            pl.BlockSpec((_D, _DP), lambda i: (0, 0)),
            pl.BlockSpec((1, _DP), lambda i: (0, 0)),
        ],
        out_specs=[
            pl.BlockSpec((blk, _DP), lambda i: (i, 0)),
            pl.BlockSpec((blk, _DP), lambda i: (i, 0)),
        ],
        out_shape=[
            jax.ShapeDtypeStruct((n, _DP), jnp.float32),
            jax.ShapeDtypeStruct((n, _DP), jnp.float32),
        ],
    )(node_feat, wa, ba, wb, bb)


def _node_update(node_feat, ah, num, den, blk):
    n = node_feat.shape[0]
    grid = n // blk
    return pl.pallas_call(
        _node_body,
        grid=(grid,),
        in_specs=[
            _row_spec(blk),
            pl.BlockSpec((blk, _DP), lambda i: (i, 0)),
            pl.BlockSpec((blk, _DP), lambda i: (i, 0)),
            pl.BlockSpec((blk, _DP), lambda i: (i, 0)),
        ],
        out_specs=[
            pl.BlockSpec((blk, _DP), lambda i: (i, 0)),
            pl.BlockSpec((8, _DP), lambda i: (0, 0)),
        ],
        out_shape=[
            jax.ShapeDtypeStruct((n, _DP), jnp.float32),
            jax.ShapeDtypeStruct((8, _DP), jnp.float32),
        ],
    )(node_feat, ah, num, den)


def _gru_chain(e, sh, dh, ws, bs, blk):
    m = e.shape[0]
    grid = m // blk
    return pl.pallas_call(
        _gru_body,
        grid=(grid,),
        in_specs=[
            _row_spec(blk),
            pl.BlockSpec((blk, _DP), lambda i: (i, 0)),
            pl.BlockSpec((blk, _DP), lambda i: (i, 0)),
            pl.BlockSpec((12, _D, _D), lambda i: (0, 0, 0)),
            pl.BlockSpec((16, _D), lambda i: (0, 0)),
        ],
        out_specs=[_row_spec(blk), pl.BlockSpec((8, _D), lambda i: (0, 0))],
        out_shape=[
            jax.ShapeDtypeStruct((m, _D), jnp.float32),
            jax.ShapeDtypeStruct((8, _D), jnp.float32),
        ],
    )(e, sh, dh, ws, bs)


def _bn_apply(x, res, scale, shift, blk):
    m, xw = x.shape
    grid = m // blk
    return pl.pallas_call(
        _apply_body,
        grid=(grid,),
        in_specs=[
            pl.BlockSpec((blk, xw), lambda i: (i, 0)),
            _row_spec(blk),
            pl.BlockSpec((1, _D), lambda i: (0, 0)),
            pl.BlockSpec((1, _D), lambda i: (0, 0)),
        ],
        out_specs=_row_spec(blk),
        out_shape=jax.ShapeDtypeStruct((m, _D), jnp.float32),
    )(x, res, scale, shift)


def _bn_coeffs(stats, count, gamma, beta):
    mean = stats[0, :] / count
    var = stats[1, :] / count - mean * mean
    inv = jax.lax.rsqrt(var + 1e-5)
    scale = gamma * inv
    shift = beta - mean * scale
    return scale.reshape(1, _D), shift.reshape(1, _D)


def kernel(node_feat, edge_index, edge_feat, W_A, b_A, W_B, b_B,
           W_ih1, W_hh1, b_ih1, b_hh1, W_ih2, W_hh2, b_ih2, b_hh2,
           gamma_h, beta_h, gamma_e, beta_e):
    n = node_feat.shape[0]
    m = edge_feat.shape[0]
    nblk = 5000 if n % 5000 == 0 else 8
    eblk = 4000 if m % 4000 == 0 else 8

    src = edge_index[0]
    dst = edge_index[1]

    ah, bh = _compute_ab(node_feat, W_A, b_A, W_B, b_B, nblk)

    sig_pad = _sig_pad(edge_feat, eblk)
    num_full, den_full = _sc_segsum(sig_pad, bh, src, dst, m)
    num = num_full[:n]
    den = den_full[:n]

    h_pre, h_stats = _node_update(node_feat, ah, num, den, nblk)

    sh, dh = _sc_gather2(h_pre, src, dst, 200)

    def split3(w):
        return [w[0:_D].T, w[_D:2 * _D].T, w[2 * _D:3 * _D].T]

    ws = jnp.stack(split3(W_ih1) + split3(W_hh1) + split3(W_ih2) + split3(W_hh2))
    bs = jnp.concatenate([
        b_ih1.reshape(3, _D), b_hh1.reshape(3, _D),
        b_ih2.reshape(3, _D), b_hh2.reshape(3, _D),
        jnp.zeros((4, _D), jnp.float32),
    ])

    e2, e_stats = _gru_chain(edge_feat, sh, dh, ws, bs, eblk)

    h_scale, h_shift = _bn_coeffs(h_stats[:, :_D], jnp.float32(n), gamma_h, beta_h)
    e_scale, e_shift = _bn_coeffs(e_stats, jnp.float32(m), gamma_e, beta_e)

    h_out = _bn_apply(h_pre, node_feat, h_scale, h_shift, nblk)
    e_out = _bn_apply(e2, edge_feat, e_scale, e_shift, eblk)
    return (h_out, e_out)


# speedup vs baseline: 1.2881x; 1.2881x over previous
import functools

import jax
import jax.numpy as jnp
from jax import lax
from jax.experimental import pallas as pl
from jax.experimental.pallas import tpu as pltpu
from jax.experimental.pallas import tpu_sc as plsc

_D = 96
_DP = 128
_NC = 2
_NS = 16
_NW = _NC * _NS


def _sc_gather2(table, idx_a, idx_b, chunk):
    e = idx_a.shape[0]
    d = table.shape[1]
    bpw = e // _NW
    n_iter = bpw // chunk
    mesh = plsc.VectorSubcoreMesh(core_axis_name="c", subcore_axis_name="s")

    @functools.partial(
        pl.kernel,
        mesh=mesh,
        out_type=[
            jax.ShapeDtypeStruct((e, d), jnp.float32),
            jax.ShapeDtypeStruct((e, d), jnp.float32),
        ],
        scratch_types=[
            pltpu.VMEM((chunk,), jnp.int32),
            pltpu.VMEM((chunk,), jnp.int32),
            pltpu.VMEM((chunk, d), jnp.float32),
            pltpu.VMEM((chunk, d), jnp.float32),
            pltpu.SemaphoreType.DMA,
            pltpu.SemaphoreType.DMA,
        ],
    )
    def k(table_hbm, ia_hbm, ib_hbm, oa_hbm, ob_hbm,
          ia_v, ib_v, ra_v, rb_v, sem_a, sem_b):
        wid = lax.axis_index("s") * _NC + lax.axis_index("c")
        base = wid * bpw

        def body(i, _):
            off = base + i * chunk
            pltpu.sync_copy(ia_hbm.at[pl.ds(off, chunk)], ia_v)
            pltpu.sync_copy(ib_hbm.at[pl.ds(off, chunk)], ib_v)
            ca = pltpu.async_copy(table_hbm.at[ia_v], ra_v, sem_a)
            cb = pltpu.async_copy(table_hbm.at[ib_v], rb_v, sem_b)
            ca.wait()
            pltpu.sync_copy(ra_v, oa_hbm.at[pl.ds(off, chunk)])
            cb.wait()
            pltpu.sync_copy(rb_v, ob_hbm.at[pl.ds(off, chunk)])
            return 0

        lax.fori_loop(0, n_iter, body, 0)

    return k(table, idx_a, idx_b)


def _sc_gather1(table, idx, chunk):
    e = idx.shape[0]
    d = table.shape[1]
    bpw = e // _NW
    n_iter = bpw // chunk
    mesh = plsc.VectorSubcoreMesh(core_axis_name="c", subcore_axis_name="s")

    @functools.partial(
        pl.kernel,
        mesh=mesh,
        out_type=jax.ShapeDtypeStruct((e, d), jnp.float32),
        scratch_types=[
            pltpu.VMEM((chunk,), jnp.int32),
            pltpu.VMEM((chunk, d), jnp.float32),
            pltpu.SemaphoreType.DMA,
        ],
    )
    def k(table_hbm, i_hbm, o_hbm, i_v, r_v, sem):
        wid = lax.axis_index("s") * _NC + lax.axis_index("c")
        base = wid * bpw

        def body(i, _):
            off = base + i * chunk
            pltpu.sync_copy(i_hbm.at[pl.ds(off, chunk)], i_v)
            pltpu.async_copy(table_hbm.at[i_v], r_v, sem).wait()
            pltpu.sync_copy(r_v, o_hbm.at[pl.ds(off, chunk)])
            return 0

        lax.fori_loop(0, n_iter, body, 0)

    return k(table, idx)


_CN = 4352
_NCHUNK = 12
_G = 80
_BUF = 4096
_STOP = 3952
_NB = _BUF // _G


def _sc_segsum(sig_pad, bh, src, dst, e):
    n_out = _NCHUNK * _CN
    epw = e // _NS
    PF = 2000
    n_pf = epw // PF
    n_vec = PF // 16
    mesh = plsc.VectorSubcoreMesh(core_axis_name="c", subcore_axis_name="s")

    @functools.partial(
        pl.kernel,
        mesh=mesh,
        out_type=[
            jax.ShapeDtypeStruct((n_out, _DP), jnp.float32),
            jax.ShapeDtypeStruct((n_out, _DP), jnp.float32),
        ],
        scratch_types=[
            pltpu.VMEM((PF,), jnp.int32),
            pltpu.VMEM((PF,), jnp.int32),
            pltpu.VMEM((_BUF,), jnp.int32),
            pltpu.VMEM((_BUF,), jnp.int32),
            pltpu.VMEM((_BUF,), jnp.int32),
            pltpu.VMEM((_NB, _G), jnp.int32),
            pltpu.VMEM((_G, _DP), jnp.float32),
            pltpu.VMEM((_G, _DP), jnp.float32),
            pltpu.VMEM((_G, _DP), jnp.float32),
            pltpu.VMEM((_G, _DP), jnp.float32),
            pltpu.VMEM((16, _DP), jnp.float32),
            pltpu.VMEM_SHARED((_CN + 16, _DP), jnp.float32),
            pltpu.VMEM_SHARED((_CN + 16, _DP), jnp.float32),
            pltpu.SemaphoreType.DMA,
            pltpu.SemaphoreType.DMA,
            pltpu.SemaphoreType.DMA,
            pltpu.SemaphoreType.DMA,
        ],
        compiler_params=pltpu.CompilerParams(needs_layout_passes=False),
    )
    def k(sig_hbm, bh_hbm, src_hbm, dst_hbm, num_hbm, den_hbm,
          dst_pf, src_pf, eid_b, src_b, dl_b, dl2d,
          sig0, sig1, bh0, bh1, zbuf,
          acc_num, acc_den, sem_s0, sem_s1, sem_b0, sem_b1):
        c = lax.axis_index("c")
        t = lax.axis_index("s")
        ebase = t * epw
        iota = lax.iota(jnp.int32, 16)

        def _z(r, _):
            for v in range(_DP // 16):
                zbuf[r, pl.ds(v * 16, 16)] = jnp.zeros((16,), jnp.float32)
            return 0
        lax.fori_loop(0, 16, _z, 0)

        flush_rows = _CN // _NS

        nbpf = PF // _G

        def one_pass(p, _):
            q = 2 * p + c
            lo = q * _CN
            hi = lo + _CN

            zb = t * flush_rows

            def zstep(zi, _):
                pltpu.sync_copy(zbuf, acc_num.at[pl.ds(zb + zi * 16, 16)])
                pltpu.sync_copy(zbuf, acc_den.at[pl.ds(zb + zi * 16, 16)])
                return 0

            lax.fori_loop(0, flush_rows // 16, zstep, 0)
            plsc.subcore_barrier()

            def pf_step(op, _):
                eb = ebase + op * PF
                pltpu.sync_copy(dst_hbm.at[pl.ds(eb, PF)], dst_pf)
                pltpu.sync_copy(src_hbm.at[pl.ds(eb, PF)], src_pf)

                def vstep(kk, cnt):
                    sl = pl.ds(kk * 16, 16)
                    dv = dst_pf[sl]
                    sv = src_pf[sl]
                    m = (dv >= lo) & (dv < hi)
                    dlv = jnp.where(m, dv - lo, jnp.full((16,), _CN, jnp.int32))
                    ev = eb + kk * 16 + iota
                    pk = sv * 8192 + dlv
                    key = iota + jnp.where(m, jnp.int32(0), jnp.int32(16))
                    _, ev_s = plsc.sort_key_val(key, ev)
                    _, pk_s = plsc.sort_key_val(key, pk)
                    eid_b[pl.ds(cnt, 16)] = ev_s
                    dl_b[pl.ds(cnt, 16)] = pk_s
                    ones = jnp.where(m, jnp.int32(1), jnp.int32(0))
                    return cnt + jnp.sum(ones)

                cnt = lax.fori_loop(0, n_vec, vstep, jnp.int32(0))

                for pi in range(_G // 16):
                    eid_b[pl.ds(cnt + pi * 16, 16)] = jnp.zeros((16,), jnp.int32)
                    dl_b[pl.ds(cnt + pi * 16, 16)] = jnp.full((16,), _CN, jnp.int32)
                nb = (cnt + _G - 1) // _G

                def bstep(j, _):
                    for v in range(_G // 16):
                        pk16 = dl_b[pl.ds(j * _G + v * 16, 16)]
                        src_b[pl.ds(v * 16, 16)] = pk16 // 8192
                        dl2d[0, pl.ds(v * 16, 16)] = pk16 % 8192
                    cs = pltpu.async_copy(
                        sig_hbm.at[eid_b.at[pl.ds(j * _G, _G)]], sig0, sem_s0)
                    cb = pltpu.async_copy(
                        bh_hbm.at[src_b.at[pl.ds(0, _G)]], bh0, sem_b0)
                    cs.wait()
                    cb.wait()

                    def mul_row(r, _):
                        for v in range(_DP // 16):
                            sl = pl.ds(v * 16, 16)
                            bh0[r, sl] = bh0[r, sl] * sig0[r, sl]
                        return 0
                    lax.fori_loop(0, _G, mul_row, 0)
                    pltpu.sync_copy(bh0, acc_num.at[dl2d.at[0]], add=True)
                    pltpu.sync_copy(sig0, acc_den.at[dl2d.at[0]], add=True)
                    return 0

                lax.fori_loop(0, nb, bstep, 0)
                return 0

            lax.fori_loop(0, n_pf, pf_step, 0)
            plsc.subcore_barrier()

            fb = t * flush_rows

            def fstep(fi, _):
                pltpu.sync_copy(acc_num.at[pl.ds(fb + fi * 16, 16)], zbuf)
                pltpu.sync_copy(zbuf, num_hbm.at[pl.ds(lo + fb + fi * 16, 16)])
                pltpu.sync_copy(acc_den.at[pl.ds(fb + fi * 16, 16)], zbuf)
                pltpu.sync_copy(zbuf, den_hbm.at[pl.ds(lo + fb + fi * 16, 16)])
                return 0

            lax.fori_loop(0, flush_rows // 16, fstep, 0)

            def _rz(r, _):
                for v in range(_DP // 16):
                    zbuf[r, pl.ds(v * 16, 16)] = jnp.zeros((16,), jnp.float32)
                return 0
            lax.fori_loop(0, 16, _rz, 0)
            plsc.subcore_barrier()
            return 0

        lax.fori_loop(0, _NCHUNK // _NC, one_pass, 0)

    return k(sig_pad, bh, src, dst)


def _ab_body(x_ref, wa_ref, ba_ref, wb_ref, bb_ref, ah_ref, bh_ref):
    x = x_ref[...]
    ah_ref[...] = jnp.dot(x, wa_ref[...], preferred_element_type=jnp.float32) + ba_ref[...]
    bh_ref[...] = jnp.dot(x, wb_ref[...], preferred_element_type=jnp.float32) + bb_ref[...]


def _sig_body(e_ref, s_ref):
    pad = jnp.zeros((e_ref.shape[0], _DP - _D), jnp.float32)
    s_ref[...] = jnp.concatenate(
        [jax.nn.sigmoid(e_ref[...]), pad], axis=1)


def _sig_pad(edge_feat, blk):
    m = edge_feat.shape[0]
    return pl.pallas_call(
        _sig_body,
        grid=(m // blk,),
        in_specs=[_row_spec(blk)],
        out_specs=pl.BlockSpec((blk, _DP), lambda i: (i, 0)),
        out_shape=jax.ShapeDtypeStruct((m, _DP), jnp.float32),
    )(edge_feat)


def _node_body(nf_ref, ah_ref, num_ref, den_ref, h_ref, stats_ref):
    i = pl.program_id(0)
    den = den_ref[...]
    pad = jnp.zeros((nf_ref.shape[0], _DP - _D), jnp.float32)
    h_agg = ah_ref[...] + num_ref[...] / (den + 1e-6)
    mask = den[:, 0:1] > 0.0
    h = jnp.where(mask, h_agg, jnp.concatenate([nf_ref[...], pad], axis=1))
    h_ref[...] = h

    @pl.when(i == 0)
    def _():
        stats_ref[...] = jnp.zeros_like(stats_ref)

    stats_ref[0:1, :] += jnp.sum(h, axis=0, keepdims=True)
    stats_ref[1:2, :] += jnp.sum(h * h, axis=0, keepdims=True)


def _gru_body(e_ref, sh_ref, dh_ref, w_ref, b_ref, e2_ref, stats_ref):
    i = pl.program_id(0)
    e = e_ref[...]
    sh = sh_ref[:, :_D]
    dh = dh_ref[:, :_D]

    def mm(x, k):
        return jnp.dot(x, w_ref[k], preferred_element_type=jnp.float32)

    def bias(k):
        return b_ref[k : k + 1, :]

    r1 = jax.nn.sigmoid(mm(sh, 0) + bias(0) + mm(e, 3) + bias(3))
    z1 = jax.nn.sigmoid(mm(sh, 1) + bias(1) + mm(e, 4) + bias(4))
    n1 = jnp.tanh(mm(sh, 2) + bias(2) + r1 * (mm(e, 5) + bias(5)))
    e1 = (1.0 - z1) * n1 + z1 * e

    r2 = jax.nn.sigmoid(mm(dh, 6) + bias(6) + mm(e1, 9) + bias(9))
    z2 = jax.nn.sigmoid(mm(dh, 7) + bias(7) + mm(e1, 10) + bias(10))
    n2 = jnp.tanh(mm(dh, 8) + bias(8) + r2 * (mm(e1, 11) + bias(11)))
    e2 = (1.0 - z2) * n2 + z2 * e1

    e2_ref[...] = e2

    @pl.when(i == 0)
    def _():
        stats_ref[...] = jnp.zeros_like(stats_ref)

    stats_ref[0:1, :] += jnp.sum(e2, axis=0, keepdims=True)
    stats_ref[1:2, :] += jnp.sum(e2 * e2, axis=0, keepdims=True)


def _apply_body(x_ref, res_ref, scale_ref, shift_ref, o_ref):
    x = x_ref[:, :_D]
    o_ref[...] = res_ref[...] + jnp.maximum(
        x * scale_ref[...] + shift_ref[...], 0.0
    )


def _full_spec():
    return pl.BlockSpec(lambda i: (0, 0))


def _row_spec(blk):
    return pl.BlockSpec((blk, _D), lambda i: (i, 0))


def _compute_ab(node_feat, W_A, b_A, W_B, b_B, blk):
    n = node_feat.shape[0]
    grid = n // blk
    padw = jnp.zeros((_D, _DP - _D), jnp.float32)
    padb = jnp.zeros((1, _DP - _D), jnp.float32)
    wa = jnp.concatenate([W_A.T, padw], axis=1)
    wb = jnp.concatenate([W_B.T, padw], axis=1)
    ba = jnp.concatenate([b_A.reshape(1, _D), padb], axis=1)
    bb = jnp.concatenate([b_B.reshape(1, _D), padb], axis=1)
    return pl.pallas_call(
        _ab_body,
        grid=(grid,),
        in_specs=[
            _row_spec(blk),
            pl.BlockSpec((_D, _DP), lambda i: (0, 0)),
            pl.BlockSpec((1, _DP), lambda i: (0, 0)),
            pl.BlockSpec((_D, _DP), lambda i: (0, 0)),
            pl.BlockSpec((1, _DP), lambda i: (0, 0)),
        ],
        out_specs=[
            pl.BlockSpec((blk, _DP), lambda i: (i, 0)),
            pl.BlockSpec((blk, _DP), lambda i: (i, 0)),
        ],
        out_shape=[
            jax.ShapeDtypeStruct((n, _DP), jnp.float32),
            jax.ShapeDtypeStruct((n, _DP), jnp.float32),
        ],
    )(node_feat, wa, ba, wb, bb)


def _node_update(node_feat, ah, num, den, blk):
    n = node_feat.shape[0]
    grid = n // blk
    return pl.pallas_call(
        _node_body,
        grid=(grid,),
        in_specs=[
            _row_spec(blk),
            pl.BlockSpec((blk, _DP), lambda i: (i, 0)),
            pl.BlockSpec((blk, _DP), lambda i: (i, 0)),
            pl.BlockSpec((blk, _DP), lambda i: (i, 0)),
        ],
        out_specs=[
            pl.BlockSpec((blk, _DP), lambda i: (i, 0)),
            pl.BlockSpec((8, _DP), lambda i: (0, 0)),
        ],
        out_shape=[
            jax.ShapeDtypeStruct((n, _DP), jnp.float32),
            jax.ShapeDtypeStruct((8, _DP), jnp.float32),
        ],
    )(node_feat, ah, num, den)


def _gru_chain(e, sh, dh, ws, bs, blk):
    m = e.shape[0]
    grid = m // blk
    return pl.pallas_call(
        _gru_body,
        grid=(grid,),
        in_specs=[
            _row_spec(blk),
            pl.BlockSpec((blk, _DP), lambda i: (i, 0)),
            pl.BlockSpec((blk, _DP), lambda i: (i, 0)),
            pl.BlockSpec((12, _D, _D), lambda i: (0, 0, 0)),
            pl.BlockSpec((16, _D), lambda i: (0, 0)),
        ],
        out_specs=[_row_spec(blk), pl.BlockSpec((8, _D), lambda i: (0, 0))],
        out_shape=[
            jax.ShapeDtypeStruct((m, _D), jnp.float32),
            jax.ShapeDtypeStruct((8, _D), jnp.float32),
        ],
    )(e, sh, dh, ws, bs)


def _bn_apply(x, res, scale, shift, blk):
    m, xw = x.shape
    grid = m // blk
    return pl.pallas_call(
        _apply_body,
        grid=(grid,),
        in_specs=[
            pl.BlockSpec((blk, xw), lambda i: (i, 0)),
            _row_spec(blk),
            pl.BlockSpec((1, _D), lambda i: (0, 0)),
            pl.BlockSpec((1, _D), lambda i: (0, 0)),
        ],
        out_specs=_row_spec(blk),
        out_shape=jax.ShapeDtypeStruct((m, _D), jnp.float32),
    )(x, res, scale, shift)


def _bn_coeffs(stats, count, gamma, beta):
    mean = stats[0, :] / count
    var = stats[1, :] / count - mean * mean
    inv = jax.lax.rsqrt(var + 1e-5)
    scale = gamma * inv
    shift = beta - mean * scale
    return scale.reshape(1, _D), shift.reshape(1, _D)


def kernel(node_feat, edge_index, edge_feat, W_A, b_A, W_B, b_B,
           W_ih1, W_hh1, b_ih1, b_hh1, W_ih2, W_hh2, b_ih2, b_hh2,
           gamma_h, beta_h, gamma_e, beta_e):
    n = node_feat.shape[0]
    m = edge_feat.shape[0]
    nblk = 5000 if n % 5000 == 0 else 8
    eblk = 4000 if m % 4000 == 0 else 8

    src = edge_index[0]
    dst = edge_index[1]

    ah, bh = _compute_ab(node_feat, W_A, b_A, W_B, b_B, nblk)

    sig_pad = _sig_pad(edge_feat, eblk)
    num_full, den_full = _sc_segsum(sig_pad, bh, src, dst, m)
    num = num_full[:n]
    den = den_full[:n]

    h_pre, h_stats = _node_update(node_feat, ah, num, den, nblk)

    sh, dh = _sc_gather2(h_pre, src, dst, 200)

    def split3(w):
        return [w[0:_D].T, w[_D:2 * _D].T, w[2 * _D:3 * _D].T]

    ws = jnp.stack(split3(W_ih1) + split3(W_hh1) + split3(W_ih2) + split3(W_hh2))
    bs = jnp.concatenate([
        b_ih1.reshape(3, _D), b_hh1.reshape(3, _D),
        b_ih2.reshape(3, _D), b_hh2.reshape(3, _D),
        jnp.zeros((4, _D), jnp.float32),
    ])

    e2, e_stats = _gru_chain(edge_feat, sh, dh, ws, bs, eblk)

    h_scale, h_shift = _bn_coeffs(h_stats[:, :_D], jnp.float32(n), gamma_h, beta_h)
    e_scale, e_shift = _bn_coeffs(e_stats, jnp.float32(m), gamma_e, beta_e)

    h_out = _bn_apply(h_pre, node_feat, h_scale, h_shift, nblk)
    e_out = _bn_apply(e2, edge_feat, e_scale, e_shift, eblk)
    return (h_out, e_out)
